# Initial kernel scaffold; baseline (speedup 1.0000x reference)
import jax
import jax.numpy as jnp
from jax.experimental import pallas as pl


def kernel(seq, emb_table):
    raise NotImplementedError("write your pallas kernel here")


# speedup vs baseline: 1.4768x; 1.4768x over previous
import functools

import jax
import jax.numpy as jnp
from jax import lax
from jax.experimental import pallas as pl
from jax.experimental.pallas import tpu as pltpu
from jax.experimental.pallas import tpu_sc as plsc

BATCH = 4096
SEQ_LEN = 200
EMBED_DIM = 32
B_TOTAL = BATCH * SEQ_LEN
NUM_WORKERS = 32
B_PER_W = B_TOTAL // NUM_WORKERS
CHUNK = 1600
NUM_CHUNKS = B_PER_W // CHUNK


def _emb_kernel(table_hbm, idx_hbm, out_hbm, idx_v, rows_v, sem):
    wid = lax.axis_index("s") * 2 + lax.axis_index("c")
    base = wid * B_PER_W

    def body(i, carry):
        off = base + i * CHUNK
        pltpu.sync_copy(idx_hbm.at[pl.ds(off, CHUNK)], idx_v)
        pltpu.async_copy(table_hbm.at[idx_v], rows_v, sem).wait()
        pltpu.sync_copy(rows_v, out_hbm.at[pl.ds(off, CHUNK)])
        return carry

    lax.fori_loop(0, NUM_CHUNKS, body, 0)


@jax.jit
def kernel(seq, emb_table):
    flat_idx = seq.reshape(B_TOTAL)
    call = pl.kernel(
        _emb_kernel,
        out_type=jax.ShapeDtypeStruct((B_TOTAL, EMBED_DIM), jnp.float32),
        mesh=plsc.VectorSubcoreMesh(core_axis_name="c", subcore_axis_name="s"),
        scratch_types=[
            pltpu.VMEM((CHUNK,), jnp.int32),
            pltpu.VMEM((CHUNK, EMBED_DIM), jnp.float32),
            pltpu.SemaphoreType.DMA,
        ],
        compiler_params=pltpu.CompilerParams(use_tc_tiling_on_sc=False),
    )
    out = call(emb_table, flat_idx)
    return out.reshape(BATCH, SEQ_LEN, EMBED_DIM)


# speedup vs baseline: 1.5008x; 1.0162x over previous
import jax
import jax.numpy as jnp
from jax import lax
from jax.experimental import pallas as pl
from jax.experimental.pallas import tpu as pltpu
from jax.experimental.pallas import tpu_sc as plsc

BATCH = 4096
SEQ_LEN = 200
EMBED_DIM = 32
B_TOTAL = BATCH * SEQ_LEN
NUM_WORKERS = 32
B_PER_W = B_TOTAL // NUM_WORKERS
CHUNK = 800
NUM_CHUNKS = B_PER_W // CHUNK
NBUF = 4


def _emb_kernel(table_hbm, idx_hbm, out_hbm, idx_v, rows_v, *sems):
    sems_i = sems[0:NBUF]
    sems_g = sems[NBUF:2 * NBUF]
    sems_o = sems[2 * NBUF:3 * NBUF]

    wid = lax.axis_index("s") * 2 + lax.axis_index("c")
    base = wid * B_PER_W
    last = NUM_CHUNKS - 1

    def off(i):
        return base + i * CHUNK

    def start_idx(i, b):
        pltpu.async_copy(idx_hbm.at[pl.ds(off(i), CHUNK)], idx_v.at[b], sems_i[b])

    def wait_idx(i, b):
        pltpu.make_async_copy(
            idx_hbm.at[pl.ds(off(i), CHUNK)], idx_v.at[b], sems_i[b]).wait()

    def start_gather(b):
        pltpu.async_copy(table_hbm.at[idx_v.at[b]], rows_v.at[b], sems_g[b])

    def wait_gather(b):
        pltpu.make_async_copy(
            table_hbm.at[idx_v.at[b]], rows_v.at[b], sems_g[b]).wait()

    def start_wb(i, b):
        pltpu.async_copy(rows_v.at[b], out_hbm.at[pl.ds(off(i), CHUNK)], sems_o[b])

    def wait_wb(i, b):
        pltpu.make_async_copy(
            rows_v.at[b], out_hbm.at[pl.ds(off(i), CHUNK)], sems_o[b]).wait()

    for b in range(NBUF):
        start_idx(b, b)

    wait_idx(0, 0)
    start_gather(0)
    for b in range(1, NBUF):
        wait_idx(b, b)
        start_gather(b)
        bp = b - 1
        wait_gather(bp)
        start_wb(bp, bp)
        start_idx(bp + NBUF, bp)

    def group(k, carry):
        for b in range(NBUF):
            i = k * NBUF + b
            bp = (b - 1) % NBUF
            wait_idx(i, b)
            wait_wb(i - NBUF, b)
            start_gather(b)
            wait_gather(bp)
            start_wb(i - 1, bp)
            start_idx(jnp.minimum(i - 1 + NBUF, last), bp)
        return carry

    lax.fori_loop(1, NUM_CHUNKS // NBUF, group, 0)

    bl = last % NBUF
    wait_gather(bl)
    start_wb(last, bl)
    for j in range(NUM_CHUNKS - NBUF, NUM_CHUNKS):
        wait_wb(j, j % NBUF)
    for j in range(NUM_CHUNKS, NUM_CHUNKS + NBUF - 1):
        wait_idx(last, j % NBUF)


@jax.jit
def kernel(seq, emb_table):
    flat_idx = seq.reshape(B_TOTAL)
    call = pl.kernel(
        _emb_kernel,
        out_type=jax.ShapeDtypeStruct((B_TOTAL, EMBED_DIM), jnp.float32),
        mesh=plsc.VectorSubcoreMesh(core_axis_name="c", subcore_axis_name="s"),
        scratch_types=[
            pltpu.VMEM((NBUF, CHUNK), jnp.int32),
            pltpu.VMEM((NBUF, CHUNK, EMBED_DIM), jnp.float32),
        ] + [pltpu.SemaphoreType.DMA] * (3 * NBUF),
        compiler_params=pltpu.CompilerParams(use_tc_tiling_on_sc=False),
    )
    out = call(emb_table, flat_idx)
    return out.reshape(BATCH, SEQ_LEN, EMBED_DIM)
